# Initial kernel scaffold; baseline (speedup 1.0000x reference)
import jax
import jax.numpy as jnp
from jax.experimental import pallas as pl


def kernel(eb_inputs, mlp_inputs, W_eb0, W_eb1, bW0, bb0, bW1, bb1, bW2, bb2, tW0, tb0, tW1, tb1, tW2, tb2, tW3, tb3):
    raise NotImplementedError("write your pallas kernel here")


# speedup vs baseline: 590.5275x; 590.5275x over previous
import functools

import jax
import jax.numpy as jnp
from jax.experimental import pallas as pl
from jax.experimental.pallas import tpu as pltpu


_BR = 2048


def _reduce_body(eb_ref, s_ref, q_ref):
    x = eb_ref[...]
    s_ref[...] = jnp.sum(x, axis=1)
    q_ref[...] = jnp.sum((x == 2).astype(jnp.int32), axis=1)


def _bf(x):
    u = jax.lax.bitcast_convert_type(x, jnp.int32)
    bias = jnp.int32(0x7FFF) + (jax.lax.shift_right_logical(u, 16) & 1)
    u = (u + bias) & jnp.int32(-65536)
    return jax.lax.bitcast_convert_type(u, jnp.float32)


def _tower_body(L, s_ref, q_ref, x0_ref, x1_ref,
                w_eb0, w_eb1, bw0, bb0, bw1, bb1, bw2, bb2,
                tw0, tb0, tw1, tb1, tw2, tb2, tw3, tb3, o_ref):
    relu = jax.nn.relu
    s = s_ref[...].astype(jnp.float32)
    q = q_ref[...].astype(jnp.float32)
    c2 = q
    c1 = s - 2.0 * q
    c0 = float(L) - c1 - c2
    x0 = _bf(x0_ref[...])
    x1 = _bf(x1_ref[...])

    h = [_bf(relu(bw0[o, 0] * x0 + bw0[o, 1] * x1 + bb0[o])) for o in range(4)]
    h2 = [_bf(relu(sum(bw1[o, k] * h[k] for k in range(4)) + bb1[o]))
          for o in range(4)]
    m = [relu(sum(bw2[o, k] * h2[k] for k in range(4)) + bb2[o])
         for o in range(3)]
    eb0 = [w_eb0[0, j] * c0 + w_eb0[1, j] * c1 + w_eb0[2, j] * c2
           for j in range(3)]
    eb1 = [w_eb1[0, j] * c0 + w_eb1[1, j] * c1 + w_eb1[2, j] * c2
           for j in range(3)]
    mb = [_bf(v) for v in m]
    feat = mb + [_bf(v) for v in eb0 + eb1] + mb
    u = [_bf(relu(sum(tw0[o, k] * feat[k] for k in range(12)) + tb0[o]))
         for o in range(4)]
    v = [_bf(relu(sum(tw1[o, k] * u[k] for k in range(4)) + tb1[o]))
         for o in range(2)]
    w = [_bf(relu(sum(tw2[o, k] * v[k] for k in range(2)) + tb2[o]))
         for o in range(2)]
    z = jax.nn.sigmoid(tw3[0, 0] * w[0] + tw3[0, 1] * w[1] + tb3[0])
    o_ref[0, 0] = 3.0 * jnp.sum(z)


def kernel(eb_inputs, mlp_inputs, W_eb0, W_eb1, bW0, bb0, bW1, bb1, bW2, bb2,
           tW0, tb0, tW1, tb1, tW2, tb2, tW3, tb3):
    B, L = eb_inputs.shape
    s, q = pl.pallas_call(
        _reduce_body,
        grid=(B // _BR,),
        in_specs=[pl.BlockSpec((_BR, L), lambda i: (i, 0))],
        out_specs=[pl.BlockSpec((_BR,), lambda i: (i,)),
                   pl.BlockSpec((_BR,), lambda i: (i,))],
        out_shape=[jax.ShapeDtypeStruct((B,), jnp.int32),
                   jax.ShapeDtypeStruct((B,), jnp.int32)],
    )(eb_inputs)

    n_lanes = 128
    rows = B // n_lanes
    s2 = s.reshape(rows, n_lanes)
    q2 = q.reshape(rows, n_lanes)
    x0 = mlp_inputs[:, 0].reshape(rows, n_lanes)
    x1 = mlp_inputs[:, 1].reshape(rows, n_lanes)

    bW0r, bW1r, bW2r = (_bf(w) for w in (bW0, bW1, bW2))
    tW0r, tW1r, tW2r, tW3r = (_bf(w) for w in (tW0, tW1, tW2, tW3))

    smem = pl.BlockSpec(memory_space=pltpu.SMEM)
    vmem = pl.BlockSpec(memory_space=pltpu.VMEM)
    out = pl.pallas_call(
        functools.partial(_tower_body, L),
        in_specs=[vmem, vmem, vmem, vmem] + [smem] * 16,
        out_specs=pl.BlockSpec(memory_space=pltpu.SMEM),
        out_shape=jax.ShapeDtypeStruct((1, 1), jnp.float32),
    )(s2, q2, x0, x1, W_eb0, W_eb1, bW0r, bb0, bW1r, bb1, bW2r, bb2,
      tW0r, tb0, tW1r, tb1, tW2r, tb2, tW3r, tb3)
    return out.reshape(())


# speedup vs baseline: 865.0802x; 1.4649x over previous
import functools

import jax
import jax.numpy as jnp
from jax.experimental import pallas as pl
from jax.experimental.pallas import tpu as pltpu


_BR = 2048


def _bf(x):
    u = jax.lax.bitcast_convert_type(x, jnp.int32)
    bias = jnp.int32(0x7FFF) + (jax.lax.shift_right_logical(u, 16) & 1)
    u = (u + bias) & jnp.int32(-65536)
    return jax.lax.bitcast_convert_type(u, jnp.float32)


def _reduce_body(L, eb_ref, s_ref, s2_ref):
    x = eb_ref[...]
    xb = x.astype(jnp.bfloat16)
    xsq = xb * xb
    n = _BR // 128
    xr = xb.reshape(n, 128, L)
    xsqr = xsq.reshape(n, 128, L)
    ones = jnp.ones((n, L), jnp.bfloat16)
    dn = (((2,), (1,)), ((0,), (0,)))
    s_ref[...] = jax.lax.dot_general(
        xr, ones, dn, preferred_element_type=jnp.float32)
    s2_ref[...] = jax.lax.dot_general(
        xsqr, ones, dn, preferred_element_type=jnp.float32)


def _tower_body(L, s_ref, s2_ref, x0_ref, x1_ref,
                w_eb0, w_eb1, bw0, bb0, bw1, bb1, bw2, bb2,
                tw0, tb0, tw1, tb1, tw2, tb2, tw3, tb3, o_ref):
    relu = jax.nn.relu
    s = s_ref[...]
    s2 = s2_ref[...]
    c2 = 0.5 * (s2 - s)
    c1 = 2.0 * s - s2
    c0 = float(L) - c1 - c2
    x0 = _bf(x0_ref[...])
    x1 = _bf(x1_ref[...])

    h = [_bf(relu(bw0[o, 0] * x0 + bw0[o, 1] * x1 + bb0[o])) for o in range(4)]
    h2 = [_bf(relu(sum(bw1[o, k] * h[k] for k in range(4)) + bb1[o]))
          for o in range(4)]
    m = [relu(sum(bw2[o, k] * h2[k] for k in range(4)) + bb2[o])
         for o in range(3)]
    eb0 = [w_eb0[0, j] * c0 + w_eb0[1, j] * c1 + w_eb0[2, j] * c2
           for j in range(3)]
    eb1 = [w_eb1[0, j] * c0 + w_eb1[1, j] * c1 + w_eb1[2, j] * c2
           for j in range(3)]
    mb = [_bf(v) for v in m]
    feat = mb + [_bf(v) for v in eb0 + eb1] + mb
    u = [_bf(relu(sum(tw0[o, k] * feat[k] for k in range(12)) + tb0[o]))
         for o in range(4)]
    v = [_bf(relu(sum(tw1[o, k] * u[k] for k in range(4)) + tb1[o]))
         for o in range(2)]
    w = [_bf(relu(sum(tw2[o, k] * v[k] for k in range(2)) + tb2[o]))
         for o in range(2)]
    z = jax.nn.sigmoid(tw3[0, 0] * w[0] + tw3[0, 1] * w[1] + tb3[0])
    o_ref[0, 0] = 3.0 * jnp.sum(z)


def kernel(eb_inputs, mlp_inputs, W_eb0, W_eb1, bW0, bb0, bW1, bb1, bW2, bb2,
           tW0, tb0, tW1, tb1, tW2, tb2, tW3, tb3):
    B, L = eb_inputs.shape
    n = _BR // 128
    s, s2 = pl.pallas_call(
        functools.partial(_reduce_body, L),
        grid=(B // _BR,),
        in_specs=[pl.BlockSpec((_BR, L), lambda i: (i, 0))],
        out_specs=[pl.BlockSpec((n, 128), lambda i: (i, 0)),
                   pl.BlockSpec((n, 128), lambda i: (i, 0))],
        out_shape=[jax.ShapeDtypeStruct((B // 128, 128), jnp.float32),
                   jax.ShapeDtypeStruct((B // 128, 128), jnp.float32)],
    )(eb_inputs)

    n_lanes = 128
    rows = B // n_lanes
    x0 = mlp_inputs[:, 0].reshape(rows, n_lanes)
    x1 = mlp_inputs[:, 1].reshape(rows, n_lanes)

    smem = pl.BlockSpec(memory_space=pltpu.SMEM)
    vmem = pl.BlockSpec(memory_space=pltpu.VMEM)
    out = pl.pallas_call(
        functools.partial(_tower_body, L),
        in_specs=[vmem, vmem, vmem, vmem] + [smem] * 16,
        out_specs=pl.BlockSpec(memory_space=pltpu.SMEM),
        out_shape=jax.ShapeDtypeStruct((1, 1), jnp.float32),
    )(s, s2, x0, x1, W_eb0, W_eb1, bW0, bb0, bW1, bb1, bW2, bb2,
      tW0, tb0, tW1, tb1, tW2, tb2, tW3, tb3)
    return out.reshape(())
